# speedup vs baseline: 4.9541x; 4.9541x over previous
import functools

import jax
import jax.numpy as jnp
from jax.experimental import pallas as pl
from jax.experimental.pallas import tpu as pltpu

NUM_BINS = 15
EPS = 1e-9
ROWS = 16384
COLS = 1000


def _tc_body(x_ref, u_ref, cal_ref, o_ref):
    x = x_ref[...]
    m = jnp.max(x, axis=1, keepdims=True)
    e = jnp.exp(x - m)
    s = jnp.sum(e, axis=1, keepdims=True)
    p = jnp.clip(e / s, EPS, 1.0 - EPS)
    acc = jnp.full(x.shape, cal_ref[0], dtype=jnp.float32)
    for j in range(NUM_BINS - 1):
        acc = acc + jnp.where(p >= u_ref[j], cal_ref[j + 1], 0.0)
    o_ref[...] = acc


def _tc_calibrate(logits, u, cal, block_rows):
    grid = logits.shape[0] // block_rows
    return pl.pallas_call(
        _tc_body,
        grid=(grid,),
        in_specs=[
            pl.BlockSpec((block_rows, COLS), lambda i: (i, 0)),
            pl.BlockSpec(memory_space=pltpu.SMEM),
            pl.BlockSpec(memory_space=pltpu.SMEM),
        ],
        out_specs=pl.BlockSpec((block_rows, COLS), lambda i: (i, 0)),
        out_shape=jax.ShapeDtypeStruct(logits.shape, jnp.float32),
    )(logits, u, cal)


@jax.jit
def kernel(logits, bin_boundaries, bin_reprs):
    u = jax.nn.sigmoid(bin_boundaries)
    t = jax.nn.sigmoid(bin_reprs)
    cal = jnp.concatenate([t[:1], jnp.diff(t)])
    return _tc_calibrate(logits, u, cal, block_rows=256)
